# Initial kernel scaffold; baseline (speedup 1.0000x reference)
import jax
import jax.numpy as jnp
from jax.experimental import pallas as pl


def kernel(x, W):
    raise NotImplementedError("write your pallas kernel here")


# speedup vs baseline: 3.2713x; 3.2713x over previous
import jax
import jax.numpy as jnp
from jax import lax
from jax.experimental import pallas as pl
from jax.experimental.pallas import tpu as pltpu
from jax.experimental.pallas import tpu_sc as plsc

D_MODEL = 128
BATCH = 4096
HIST = 50
N = BATCH * HIST

_info = plsc.get_sparse_core_info()
NC, NS = _info.num_cores, _info.num_subcores
NW = NC * NS
B_PER_W = N // NW
CHUNK = 640
NCHUNK = B_PER_W // CHUNK


def _gather_body(x_hbm, w_hbm, out_hbm, idx_v, rows_v, sem):
    wid = lax.axis_index("s") * NC + lax.axis_index("c")
    base = wid * B_PER_W

    def body(i, carry):
        off = base + i * CHUNK
        pltpu.sync_copy(x_hbm.at[pl.ds(off, CHUNK)], idx_v)
        pltpu.async_copy(w_hbm.at[idx_v], rows_v, sem).wait()
        pltpu.sync_copy(rows_v, out_hbm.at[pl.ds(off, CHUNK)])
        return carry

    lax.fori_loop(0, NCHUNK, body, 0)


def kernel(x, W):
    xf = x.reshape(-1)
    mesh = plsc.VectorSubcoreMesh(core_axis_name="c", subcore_axis_name="s")
    out = pl.kernel(
        _gather_body,
        mesh=mesh,
        out_type=jax.ShapeDtypeStruct((N, D_MODEL), jnp.float32),
        scratch_types=[
            pltpu.VMEM((CHUNK,), jnp.int32),
            pltpu.VMEM((CHUNK, D_MODEL), jnp.float32),
            pltpu.SemaphoreType.DMA,
        ],
    )(xf, W)
    return out.reshape(BATCH, HIST, D_MODEL)


# speedup vs baseline: 3.3470x; 1.0231x over previous
import jax
import jax.numpy as jnp
from jax import lax
from jax.experimental import pallas as pl
from jax.experimental.pallas import tpu as pltpu
from jax.experimental.pallas import tpu_sc as plsc

D_MODEL = 128
BATCH = 4096
HIST = 50
N = BATCH * HIST

_info = plsc.get_sparse_core_info()
NC, NS = _info.num_cores, _info.num_subcores
NW = NC * NS
B_PER_W = N // NW
CHUNK = 400
NCHUNK = B_PER_W // CHUNK


def _gather_body(x_hbm, w_hbm, out_hbm, idx0, idx1, buf0, buf1,
                 isem0, isem1, gsem0, gsem1, ssem0, ssem1):
    wid = lax.axis_index("s") * NC + lax.axis_index("c")
    base = wid * B_PER_W

    idxs = (idx0, idx1)
    bufs = (buf0, buf1)
    isems = (isem0, isem1)
    gsems = (gsem0, gsem1)
    ssems = (ssem0, ssem1)

    def icopy(i):
        return pltpu.async_copy(
            x_hbm.at[pl.ds(base + i * CHUNK, CHUNK)], idxs[i % 2],
            isems[i % 2])

    def gather(i):
        return pltpu.async_copy(w_hbm.at[idxs[i % 2]], bufs[i % 2],
                                gsems[i % 2])

    def store(i):
        return pltpu.async_copy(
            bufs[i % 2], out_hbm.at[pl.ds(base + i * CHUNK, CHUNK)],
            ssems[i % 2])

    ic = [None] * NCHUNK
    g = [None] * NCHUNK
    s = [None] * NCHUNK

    ic[0] = icopy(0)
    ic[1] = icopy(1)
    ic[0].wait()
    g[0] = gather(0)
    for i in range(NCHUNK):
        if i + 1 < NCHUNK:
            ic[i + 1].wait()
            if i >= 1:
                s[i - 1].wait()
            g[i + 1] = gather(i + 1)
        g[i].wait()
        s[i] = store(i)
        if i + 2 < NCHUNK:
            ic[i + 2] = icopy(i + 2)
    s[NCHUNK - 2].wait()
    s[NCHUNK - 1].wait()


def kernel(x, W):
    xf = x.reshape(-1)
    mesh = plsc.VectorSubcoreMesh(core_axis_name="c", subcore_axis_name="s")
    out = pl.kernel(
        _gather_body,
        mesh=mesh,
        out_type=jax.ShapeDtypeStruct((N, D_MODEL), jnp.float32),
        scratch_types=[
            pltpu.VMEM((CHUNK,), jnp.int32),
            pltpu.VMEM((CHUNK,), jnp.int32),
            pltpu.VMEM((CHUNK, D_MODEL), jnp.float32),
            pltpu.VMEM((CHUNK, D_MODEL), jnp.float32),
            pltpu.SemaphoreType.DMA,
            pltpu.SemaphoreType.DMA,
            pltpu.SemaphoreType.DMA,
            pltpu.SemaphoreType.DMA,
            pltpu.SemaphoreType.DMA,
            pltpu.SemaphoreType.DMA,
        ],
    )(xf, W)
    return out.reshape(BATCH, HIST, D_MODEL)


# speedup vs baseline: 5.8045x; 1.7343x over previous
import jax
import jax.numpy as jnp
from jax import lax
from jax.experimental import pallas as pl
from jax.experimental.pallas import tpu as pltpu
from jax.experimental.pallas import tpu_sc as plsc

D_MODEL = 128
BATCH = 4096
HIST = 50
N = BATCH * HIST

_info = plsc.get_sparse_core_info()
NC, NS = _info.num_cores, _info.num_subcores
NW = NC * NS
ROWS_PER_W = BATCH // NW
ROWS_PER_CHUNK = 8
CHUNK = ROWS_PER_CHUNK * HIST
NCHUNK = ROWS_PER_W // ROWS_PER_CHUNK


def _gather_body(x_hbm, w_hbm, out_hbm, idx0, idx1, buf0, buf1,
                 isem0, isem1, gsem0, gsem1, ssem0, ssem1):
    wid = lax.axis_index("s") * NC + lax.axis_index("c")
    base = wid * ROWS_PER_W * HIST
    row0 = wid * ROWS_PER_W

    idxs = (idx0, idx1)
    bufs = (buf0, buf1)
    isems = (isem0, isem1)
    gsems = (gsem0, gsem1)
    ssems = (ssem0, ssem1)

    def icopy(i):
        return pltpu.async_copy(
            x_hbm.at[pl.ds(base + i * CHUNK, CHUNK)], idxs[i % 2],
            isems[i % 2])

    def gather(i):
        return pltpu.async_copy(w_hbm.at[idxs[i % 2]], bufs[i % 2],
                                gsems[i % 2])

    def store(i):
        cps = []
        for k in range(ROWS_PER_CHUNK):
            cps.append(pltpu.async_copy(
                bufs[i % 2].at[pl.ds(k * HIST, HIST), :],
                out_hbm.at[row0 + i * ROWS_PER_CHUNK + k],
                ssems[i % 2]))
        return cps

    ic = [None] * NCHUNK
    g = [None] * NCHUNK
    s = [None] * NCHUNK

    ic[0] = icopy(0)
    ic[1] = icopy(1)
    ic[0].wait()
    g[0] = gather(0)
    for i in range(NCHUNK):
        if i + 1 < NCHUNK:
            ic[i + 1].wait()
            if i >= 1:
                for cp in s[i - 1]:
                    cp.wait()
            g[i + 1] = gather(i + 1)
        g[i].wait()
        s[i] = store(i)
        if i + 2 < NCHUNK:
            ic[i + 2] = icopy(i + 2)
    for cp in s[NCHUNK - 2]:
        cp.wait()
    for cp in s[NCHUNK - 1]:
        cp.wait()


def kernel(x, W):
    xf = x.reshape(-1)
    mesh = plsc.VectorSubcoreMesh(core_axis_name="c", subcore_axis_name="s")
    out = pl.kernel(
        _gather_body,
        mesh=mesh,
        out_type=jax.ShapeDtypeStruct((BATCH, HIST, D_MODEL), jnp.float32),
        scratch_types=[
            pltpu.VMEM((CHUNK,), jnp.int32),
            pltpu.VMEM((CHUNK,), jnp.int32),
            pltpu.VMEM((CHUNK, D_MODEL), jnp.float32),
            pltpu.VMEM((CHUNK, D_MODEL), jnp.float32),
            pltpu.SemaphoreType.DMA,
            pltpu.SemaphoreType.DMA,
            pltpu.SemaphoreType.DMA,
            pltpu.SemaphoreType.DMA,
            pltpu.SemaphoreType.DMA,
            pltpu.SemaphoreType.DMA,
        ],
    )(xf, W)
    return out


# speedup vs baseline: 10.3576x; 1.7844x over previous
import jax
import jax.numpy as jnp
from jax import lax
from jax.experimental import pallas as pl
from jax.experimental.pallas import tpu as pltpu
from jax.experimental.pallas import tpu_sc as plsc

D_MODEL = 128
BATCH = 4096
HIST = 50
N = BATCH * HIST

_info = plsc.get_sparse_core_info()
NC, NS = _info.num_cores, _info.num_subcores
NW = NC * NS
B_PER_W = N // NW
CHUNK = 400
NCHUNK = B_PER_W // CHUNK


def _gather_body(x_hbm, w_hbm, out_hbm, idx0, idx1, buf0, buf1,
                 isem0, isem1, gsem0, gsem1, ssem0, ssem1):
    wid = lax.axis_index("s") * NC + lax.axis_index("c")
    base = wid * B_PER_W

    idxs = (idx0, idx1)
    bufs = (buf0, buf1)
    isems = (isem0, isem1)
    gsems = (gsem0, gsem1)
    ssems = (ssem0, ssem1)

    def icopy(i):
        return pltpu.async_copy(
            x_hbm.at[pl.ds(base + i * CHUNK, CHUNK)], idxs[i % 2],
            isems[i % 2])

    def gather(i):
        return pltpu.async_copy(w_hbm.at[idxs[i % 2]], bufs[i % 2],
                                gsems[i % 2])

    def store(i):
        return pltpu.async_copy(
            bufs[i % 2], out_hbm.at[pl.ds(base + i * CHUNK, CHUNK)],
            ssems[i % 2])

    ic = [None] * NCHUNK
    g = [None] * NCHUNK
    s = [None] * NCHUNK

    ic[0] = icopy(0)
    ic[1] = icopy(1)
    ic[0].wait()
    g[0] = gather(0)
    for i in range(NCHUNK):
        if i + 1 < NCHUNK:
            ic[i + 1].wait()
            if i >= 1:
                s[i - 1].wait()
            g[i + 1] = gather(i + 1)
        g[i].wait()
        s[i] = store(i)
        if i + 2 < NCHUNK:
            ic[i + 2] = icopy(i + 2)
    s[NCHUNK - 2].wait()
    s[NCHUNK - 1].wait()


def kernel(x, W):
    xf = x.T.reshape(-1)
    mesh = plsc.VectorSubcoreMesh(core_axis_name="c", subcore_axis_name="s")
    out = pl.kernel(
        _gather_body,
        mesh=mesh,
        out_type=jax.ShapeDtypeStruct((N, D_MODEL), jnp.float32),
        scratch_types=[
            pltpu.VMEM((CHUNK,), jnp.int32),
            pltpu.VMEM((CHUNK,), jnp.int32),
            pltpu.VMEM((CHUNK, D_MODEL), jnp.float32),
            pltpu.VMEM((CHUNK, D_MODEL), jnp.float32),
            pltpu.SemaphoreType.DMA,
            pltpu.SemaphoreType.DMA,
            pltpu.SemaphoreType.DMA,
            pltpu.SemaphoreType.DMA,
            pltpu.SemaphoreType.DMA,
            pltpu.SemaphoreType.DMA,
        ],
    )(xf, W)
    return out.reshape(HIST, BATCH, D_MODEL).transpose(1, 0, 2)
